# baseline (speedup 1.0000x reference)
import functools

import jax
import jax.numpy as jnp
from jax import lax
from jax.experimental import pallas as pl
from jax.experimental.pallas import tpu as pltpu
from jax.experimental.pallas import tpu_sc as plsc

EMBED_DIM = 1536
BATCH = 16384
K = 4


@jax.jit
def _broadcast_row(idx_arr, table):
    info = plsc.get_sparse_core_info()
    nw = info.num_cores * info.num_subcores
    b_per_w = BATCH // nw
    n_chunks = b_per_w // K
    n_vecs = EMBED_DIM // 16

    mesh = plsc.VectorSubcoreMesh(core_axis_name="c", subcore_axis_name="s")

    @functools.partial(
        pl.kernel,
        mesh=mesh,
        out_type=jax.ShapeDtypeStruct((BATCH, 1, EMBED_DIM), jnp.float32),
        scratch_types=[
            pltpu.VMEM((1,), jnp.int32),
            pltpu.VMEM((1, EMBED_DIM), jnp.float32),
            pltpu.VMEM((K, EMBED_DIM), jnp.float32),
            pltpu.SemaphoreType.DMA,
            pltpu.SemaphoreType.DMA,
        ],
    )
    def k(idx_hbm, table_hbm, out_hbm, idx_v, row_v, buf_v, gsem, ssem):
        wid = lax.axis_index("s") * info.num_cores + lax.axis_index("c")
        base = wid * b_per_w
        pltpu.sync_copy(idx_hbm, idx_v)
        pltpu.async_copy(table_hbm.at[idx_v], row_v, gsem).wait()
        for c in range(n_vecs):
            chunk = row_v[0, pl.ds(c * 16, 16)]
            for r in range(K):
                buf_v[r, pl.ds(c * 16, 16)] = chunk
        for c in range(n_chunks):
            pltpu.async_copy(buf_v, out_hbm.at[pl.ds(base + c * K, K), 0], ssem)
        for c in range(n_chunks):
            pltpu.make_async_copy(
                buf_v, out_hbm.at[pl.ds(base + c * K, K), 0], ssem
            ).wait()

    return k(idx_arr, table)


def kernel(duration_s, batch_size, table):
    idx = jnp.asarray(duration_s * 10).astype(jnp.int32)
    idx = idx + (jnp.asarray(batch_size).astype(jnp.int32) - BATCH)
    idx_arr = jnp.full((1,), idx, dtype=jnp.int32)
    return _broadcast_row(idx_arr, table)


# speedup vs baseline: 1.0818x; 1.0818x over previous
import functools

import jax
import jax.numpy as jnp
from jax import lax
from jax.experimental import pallas as pl
from jax.experimental.pallas import tpu as pltpu
from jax.experimental.pallas import tpu_sc as plsc

EMBED_DIM = 1536
BATCH = 16384
K = 8


@jax.jit
def _broadcast_row(idx_arr, table):
    info = plsc.get_sparse_core_info()
    nw = info.num_cores * info.num_subcores
    b_per_w = BATCH // nw
    n_chunks = b_per_w // K
    n_vecs = EMBED_DIM // 16

    mesh = plsc.VectorSubcoreMesh(core_axis_name="c", subcore_axis_name="s")

    @functools.partial(
        pl.kernel,
        mesh=mesh,
        out_type=jax.ShapeDtypeStruct((BATCH, 1, EMBED_DIM), jnp.float32),
        scratch_types=[
            pltpu.VMEM((1,), jnp.int32),
            pltpu.VMEM((1, EMBED_DIM), jnp.float32),
            pltpu.VMEM((K, EMBED_DIM), jnp.float32),
            pltpu.SemaphoreType.DMA,
            pltpu.SemaphoreType.DMA,
        ],
    )
    def k(idx_hbm, table_hbm, out_hbm, idx_v, row_v, buf_v, gsem, ssem):
        wid = lax.axis_index("s") * info.num_cores + lax.axis_index("c")
        base = wid * b_per_w
        pltpu.sync_copy(idx_hbm, idx_v)
        pltpu.async_copy(table_hbm.at[idx_v], row_v, gsem).wait()
        for c in range(n_vecs):
            chunk = row_v[0, pl.ds(c * 16, 16)]
            for r in range(K):
                buf_v[r, pl.ds(c * 16, 16)] = chunk
        for c in range(n_chunks):
            pltpu.async_copy(buf_v, out_hbm.at[pl.ds(base + c * K, K), 0], ssem)
        for c in range(n_chunks):
            pltpu.make_async_copy(
                buf_v, out_hbm.at[pl.ds(base + c * K, K), 0], ssem
            ).wait()

    return k(idx_arr, table)


def kernel(duration_s, batch_size, table):
    idx = jnp.asarray(duration_s * 10).astype(jnp.int32)
    idx = idx + (jnp.asarray(batch_size).astype(jnp.int32) - BATCH)
    idx_arr = jnp.full((1,), idx, dtype=jnp.int32)
    return _broadcast_row(idx_arr, table)


# speedup vs baseline: 1.0980x; 1.0150x over previous
import functools

import jax
import jax.numpy as jnp
from jax import lax
from jax.experimental import pallas as pl
from jax.experimental.pallas import tpu as pltpu
from jax.experimental.pallas import tpu_sc as plsc

EMBED_DIM = 1536
BATCH = 16384
K = 8


@jax.jit
def _broadcast_row(idx_arr, table):
    info = plsc.get_sparse_core_info()
    nw = info.num_cores * info.num_subcores
    b_per_w = BATCH // nw
    n_chunks = b_per_w // K
    n_vecs = EMBED_DIM // 16

    mesh = plsc.VectorSubcoreMesh(core_axis_name="c", subcore_axis_name="s")

    @functools.partial(
        pl.kernel,
        mesh=mesh,
        out_type=jax.ShapeDtypeStruct((BATCH, 1, EMBED_DIM), jnp.float32),
        scratch_types=[
            pltpu.VMEM((1,), jnp.int32),
            pltpu.VMEM((1, EMBED_DIM), jnp.float32),
            pltpu.VMEM((K, EMBED_DIM), jnp.float32),
            pltpu.SemaphoreType.DMA,
            pltpu.SemaphoreType.DMA,
        ],
    )
    def k(idx_hbm, table_hbm, out_hbm, idx_v, row_v, buf_v, gsem, ssem):
        wid = lax.axis_index("s") * info.num_cores + lax.axis_index("c")
        base = wid * b_per_w
        pltpu.sync_copy(idx_hbm, idx_v)
        pltpu.async_copy(table_hbm.at[idx_v], row_v, gsem).wait()
        for c in range(n_vecs):
            chunk = row_v[0, pl.ds(c * 16, 16)]
            for r in range(K):
                buf_v[r, pl.ds(c * 16, 16)] = chunk
        for c in range(n_chunks):
            pltpu.async_copy(buf_v, out_hbm.at[pl.ds(base + c * K, K), 0], ssem)
        pltpu.make_async_copy(
            out_hbm.at[pl.ds(base, b_per_w), 0],
            out_hbm.at[pl.ds(base, b_per_w), 0],
            ssem,
        ).wait()

    return k(idx_arr, table)


def kernel(duration_s, batch_size, table):
    idx = jnp.asarray(duration_s * 10).astype(jnp.int32)
    idx = idx + (jnp.asarray(batch_size).astype(jnp.int32) - BATCH)
    idx_arr = jnp.full((1,), idx, dtype=jnp.int32)
    return _broadcast_row(idx_arr, table)
